# baseline (speedup 1.0000x reference)
import jax
import jax.numpy as jnp
from jax.experimental import pallas as pl
from jax.experimental.pallas import tpu as pltpu

_ALPHA = 1.0
_B, _N, _C = 32, 8732, 81
_TOT = _B * _N
_BG = 8
_NB = 256
_GB = _B // _BG
_GN = (_N + _NB - 1) // _NB
_NP = _GN * _NB


def _body(lt_ref, tc_ref, tbbt_ref, pbbt_ref,
          total_ref, locout_ref, mined_ref,
          negce_ref, loc_ref, npos_ref, possum_ref):
    ib = pl.program_id(0)
    jn = pl.program_id(1)
    lt = lt_ref[...]
    tc = tc_ref[...]
    n_iota = jax.lax.broadcasted_iota(jnp.int32, (_BG, _NB), 1)
    valid = (n_iota + jn * _NB) < _N

    m = jnp.max(lt, axis=0)
    cls_iota = jax.lax.broadcasted_iota(jnp.int32, (_C, _BG, _NB), 0)
    sel = cls_iota == tc[None]
    s = jnp.sum(jnp.exp(lt - m[None]), axis=0)
    tgt = jnp.sum(jnp.where(sel, lt, 0.0), axis=0)
    ce = m + jnp.log(s) - tgt

    pos = (tc > 0) & valid
    negce_ref[pl.ds(ib * _BG, _BG), pl.ds(jn * _NB, _NB)] = jnp.where(
        pos | (~valid), jnp.float32(-1e30), ce)

    posf = pos.astype(jnp.float32)
    d = pbbt_ref[...] - tbbt_ref[...]
    ad = jnp.abs(d)
    sl1 = jnp.where(ad < 1.0, 0.5 * ad * ad, ad - 0.5)
    loc_part = jnp.sum(jnp.where(pos[:, None, :], sl1, 0.0))

    @pl.when((ib == 0) & (jn == 0))
    def _():
        loc_ref[0] = 0.0
        npos_ref[0] = 0.0
        possum_ref[0] = 0.0

    loc_ref[0] += loc_part
    npos_ref[0] += jnp.sum(posf)
    possum_ref[0] += jnp.sum(jnp.where(pos, ce, 0.0))

    @pl.when((ib == _GB - 1) & (jn == _GN - 1))
    def _mine():
        x = negce_ref[...]
        bits = jax.lax.bitcast_convert_type(x, jnp.int32)
        npos_raw = npos_ref[0].astype(jnp.int32)
        num_neg = _TOT - npos_raw
        npos = jnp.maximum(npos_raw, 1)
        k = jnp.minimum(npos * 3, num_neg)

        def srch_cond(c):
            b, _, exact = c
            return (b >= 0) & (exact == 0)

        def srch(c):
            b, t, _ = c
            t_try = t | (jnp.int32(1) << b)
            cnt = jnp.sum((bits >= t_try).astype(jnp.int32))
            t2 = jnp.where(cnt >= k, t_try, t)
            return (b - 1, t2, (cnt == k).astype(jnp.int32))

        _, t, _ = jax.lax.while_loop(
            srch_cond, srch, (jnp.int32(30), jnp.int32(0), jnp.int32(0)))
        gt = bits > t
        cnt_gt = jnp.sum(gt.astype(jnp.int32))
        sum_gt = jnp.sum(jnp.where(gt, x, 0.0))
        tval = jax.lax.bitcast_convert_type(t, jnp.float32)
        top = jnp.where(k > 0,
                        sum_gt + (k - cnt_gt).astype(jnp.float32) * tval,
                        jnp.float32(0.0))
        mined = (top + possum_ref[0]) / (k + npos).astype(jnp.float32)
        loc = loc_ref[0] / npos.astype(jnp.float32)
        total_ref[0, 0] = loc + _ALPHA * mined
        locout_ref[0, 0] = loc
        mined_ref[0, 0] = mined


def kernel(target_bounding_boxes, target_classes,
           predicted_bounding_boxes, predicted_class_logits):
    lt = jnp.transpose(predicted_class_logits, (2, 0, 1))
    tbbt = jnp.transpose(target_bounding_boxes, (0, 2, 1))
    pbbt = jnp.transpose(predicted_bounding_boxes, (0, 2, 1))

    s11 = jax.ShapeDtypeStruct((1, 1), jnp.float32)
    total, loc, mined = pl.pallas_call(
        _body,
        grid=(_GB, _GN),
        in_specs=[
            pl.BlockSpec((_C, _BG, _NB), lambda i, j: (0, i, j)),
            pl.BlockSpec((_BG, _NB), lambda i, j: (i, j)),
            pl.BlockSpec((_BG, 4, _NB), lambda i, j: (i, 0, j)),
            pl.BlockSpec((_BG, 4, _NB), lambda i, j: (i, 0, j)),
        ],
        out_specs=[
            pl.BlockSpec(memory_space=pltpu.SMEM),
            pl.BlockSpec(memory_space=pltpu.SMEM),
            pl.BlockSpec(memory_space=pltpu.SMEM),
        ],
        out_shape=[s11, s11, s11],
        scratch_shapes=[
            pltpu.VMEM((_B, _NP), jnp.float32),
            pltpu.SMEM((1,), jnp.float32),
            pltpu.SMEM((1,), jnp.float32),
            pltpu.SMEM((1,), jnp.float32),
        ],
    )(lt, target_classes, tbbt, pbbt)

    return total.reshape(()), loc.reshape(()), mined.reshape(())


# speedup vs baseline: 2.2714x; 2.2714x over previous
import jax
import jax.numpy as jnp
from jax.experimental import pallas as pl
from jax.experimental.pallas import tpu as pltpu

_ALPHA = 1.0
_B, _N, _C = 32, 8732, 81
_TOT = _B * _N
_BG = 16
_NB = 1024
_GB = _B // _BG
_GN = (_N + _NB - 1) // _NB
_NP = _GN * _NB


def _body(lt_ref, tc_ref, tbbt_ref, pbbt_ref,
          total_ref, locout_ref, mined_ref,
          negce_ref, loc_ref, npos_ref, possum_ref):
    ib = pl.program_id(0)
    jn = pl.program_id(1)
    lt = lt_ref[...]
    tc = tc_ref[...]
    n_iota = jax.lax.broadcasted_iota(jnp.int32, (_BG, _NB), 1)
    valid = (n_iota + jn * _NB) < _N

    m = jnp.max(lt, axis=0)
    cls_iota = jax.lax.broadcasted_iota(jnp.int32, (_C, _BG, _NB), 0)
    sel = cls_iota == tc[None]
    s = jnp.sum(jnp.exp(lt - m[None]), axis=0)
    tgt = jnp.sum(jnp.where(sel, lt, 0.0), axis=0)
    ce = m + jnp.log(s) - tgt

    pos = (tc > 0) & valid
    negce_ref[pl.ds(ib * _BG, _BG), pl.ds(jn * _NB, _NB)] = jnp.where(
        pos | (~valid), jnp.float32(-1e30), ce)

    posf = pos.astype(jnp.float32)
    d = pbbt_ref[...] - tbbt_ref[...]
    ad = jnp.abs(d)
    sl1 = jnp.where(ad < 1.0, 0.5 * ad * ad, ad - 0.5)
    loc_part = jnp.sum(jnp.where(pos[:, None, :], sl1, 0.0))

    @pl.when((ib == 0) & (jn == 0))
    def _():
        loc_ref[0] = 0.0
        npos_ref[0] = 0.0
        possum_ref[0] = 0.0

    loc_ref[0] += loc_part
    npos_ref[0] += jnp.sum(posf)
    possum_ref[0] += jnp.sum(jnp.where(pos, ce, 0.0))

    @pl.when((ib == _GB - 1) & (jn == _GN - 1))
    def _mine():
        x = negce_ref[...]
        bits = jax.lax.bitcast_convert_type(x, jnp.int32)
        npos_raw = npos_ref[0].astype(jnp.int32)
        num_neg = _TOT - npos_raw
        npos = jnp.maximum(npos_raw, 1)
        k = jnp.minimum(npos * 3, num_neg)

        def srch_cond(c):
            b, _, exact = c
            return (b >= 0) & (exact == 0)

        def srch(c):
            b, t, _ = c
            t_try = t | (jnp.int32(1) << b)
            cnt = jnp.sum((bits >= t_try).astype(jnp.int32))
            t2 = jnp.where(cnt >= k, t_try, t)
            return (b - 1, t2, (cnt == k).astype(jnp.int32))

        _, t, _ = jax.lax.while_loop(
            srch_cond, srch, (jnp.int32(30), jnp.int32(0), jnp.int32(0)))
        gt = bits > t
        cnt_gt = jnp.sum(gt.astype(jnp.int32))
        sum_gt = jnp.sum(jnp.where(gt, x, 0.0))
        tval = jax.lax.bitcast_convert_type(t, jnp.float32)
        top = jnp.where(k > 0,
                        sum_gt + (k - cnt_gt).astype(jnp.float32) * tval,
                        jnp.float32(0.0))
        mined = (top + possum_ref[0]) / (k + npos).astype(jnp.float32)
        loc = loc_ref[0] / npos.astype(jnp.float32)
        total_ref[0, 0] = loc + _ALPHA * mined
        locout_ref[0, 0] = loc
        mined_ref[0, 0] = mined


def kernel(target_bounding_boxes, target_classes,
           predicted_bounding_boxes, predicted_class_logits):
    lt = jnp.transpose(predicted_class_logits, (2, 0, 1))
    tbbt = jnp.transpose(target_bounding_boxes, (0, 2, 1))
    pbbt = jnp.transpose(predicted_bounding_boxes, (0, 2, 1))

    s11 = jax.ShapeDtypeStruct((1, 1), jnp.float32)
    total, loc, mined = pl.pallas_call(
        _body,
        grid=(_GB, _GN),
        in_specs=[
            pl.BlockSpec((_C, _BG, _NB), lambda i, j: (0, i, j)),
            pl.BlockSpec((_BG, _NB), lambda i, j: (i, j)),
            pl.BlockSpec((_BG, 4, _NB), lambda i, j: (i, 0, j)),
            pl.BlockSpec((_BG, 4, _NB), lambda i, j: (i, 0, j)),
        ],
        out_specs=[
            pl.BlockSpec(memory_space=pltpu.SMEM),
            pl.BlockSpec(memory_space=pltpu.SMEM),
            pl.BlockSpec(memory_space=pltpu.SMEM),
        ],
        out_shape=[s11, s11, s11],
        scratch_shapes=[
            pltpu.VMEM((_B, _NP), jnp.float32),
            pltpu.SMEM((1,), jnp.float32),
            pltpu.SMEM((1,), jnp.float32),
            pltpu.SMEM((1,), jnp.float32),
        ],
    )(lt, target_classes, tbbt, pbbt)

    return total.reshape(()), loc.reshape(()), mined.reshape(())


# speedup vs baseline: 2.2838x; 1.0055x over previous
import jax
import jax.numpy as jnp
from jax.experimental import pallas as pl
from jax.experimental.pallas import tpu as pltpu

_ALPHA = 1.0
_B, _N, _C = 32, 8732, 81
_TOT = _B * _N
_BG = 32
_NB = 1024
_GB = _B // _BG
_GN = (_N + _NB - 1) // _NB
_NP = _GN * _NB


def _body(lt_ref, tc_ref, tbbt_ref, pbbt_ref,
          total_ref, locout_ref, mined_ref,
          negce_ref, loc_ref, npos_ref, possum_ref):
    ib = pl.program_id(0)
    jn = pl.program_id(1)
    lt = lt_ref[...]
    tc = tc_ref[...]
    n_iota = jax.lax.broadcasted_iota(jnp.int32, (_BG, _NB), 1)
    valid = (n_iota + jn * _NB) < _N

    m = jnp.max(lt, axis=0)
    cls_iota = jax.lax.broadcasted_iota(jnp.int32, (_C, _BG, _NB), 0)
    sel = cls_iota == tc[None]
    s = jnp.sum(jnp.exp(lt - m[None]), axis=0)
    tgt = jnp.sum(jnp.where(sel, lt, 0.0), axis=0)
    ce = m + jnp.log(s) - tgt

    pos = (tc > 0) & valid
    negce_ref[pl.ds(ib * _BG, _BG), pl.ds(jn * _NB, _NB)] = jnp.where(
        pos | (~valid), jnp.float32(-1e30), ce)

    posf = pos.astype(jnp.float32)
    d = pbbt_ref[...] - tbbt_ref[...]
    ad = jnp.abs(d)
    sl1 = jnp.where(ad < 1.0, 0.5 * ad * ad, ad - 0.5)
    loc_part = jnp.sum(jnp.where(pos[:, None, :], sl1, 0.0))

    @pl.when((ib == 0) & (jn == 0))
    def _():
        loc_ref[0] = 0.0
        npos_ref[0] = 0.0
        possum_ref[0] = 0.0

    loc_ref[0] += loc_part
    npos_ref[0] += jnp.sum(posf)
    possum_ref[0] += jnp.sum(jnp.where(pos, ce, 0.0))

    @pl.when((ib == _GB - 1) & (jn == _GN - 1))
    def _mine():
        x = negce_ref[...]
        bits = jax.lax.bitcast_convert_type(x, jnp.int32)
        npos_raw = npos_ref[0].astype(jnp.int32)
        num_neg = _TOT - npos_raw
        npos = jnp.maximum(npos_raw, 1)
        k = jnp.minimum(npos * 3, num_neg)

        def srch_cond(c):
            b, _, exact = c
            return (b >= 0) & (exact == 0)

        def srch(c):
            b, t, _ = c
            t_try = t | (jnp.int32(1) << b)
            cnt = jnp.sum((bits >= t_try).astype(jnp.int32))
            t2 = jnp.where(cnt >= k, t_try, t)
            return (b - 1, t2, (cnt == k).astype(jnp.int32))

        _, t, _ = jax.lax.while_loop(
            srch_cond, srch, (jnp.int32(30), jnp.int32(0), jnp.int32(0)))
        gt = bits > t
        cnt_gt = jnp.sum(gt.astype(jnp.int32))
        sum_gt = jnp.sum(jnp.where(gt, x, 0.0))
        tval = jax.lax.bitcast_convert_type(t, jnp.float32)
        top = jnp.where(k > 0,
                        sum_gt + (k - cnt_gt).astype(jnp.float32) * tval,
                        jnp.float32(0.0))
        mined = (top + possum_ref[0]) / (k + npos).astype(jnp.float32)
        loc = loc_ref[0] / npos.astype(jnp.float32)
        total_ref[0, 0] = loc + _ALPHA * mined
        locout_ref[0, 0] = loc
        mined_ref[0, 0] = mined


def kernel(target_bounding_boxes, target_classes,
           predicted_bounding_boxes, predicted_class_logits):
    lt = jnp.transpose(predicted_class_logits, (2, 0, 1))
    tbbt = jnp.transpose(target_bounding_boxes, (0, 2, 1))
    pbbt = jnp.transpose(predicted_bounding_boxes, (0, 2, 1))

    s11 = jax.ShapeDtypeStruct((1, 1), jnp.float32)
    total, loc, mined = pl.pallas_call(
        _body,
        grid=(_GB, _GN),
        in_specs=[
            pl.BlockSpec((_C, _BG, _NB), lambda i, j: (0, i, j)),
            pl.BlockSpec((_BG, _NB), lambda i, j: (i, j)),
            pl.BlockSpec((_BG, 4, _NB), lambda i, j: (i, 0, j)),
            pl.BlockSpec((_BG, 4, _NB), lambda i, j: (i, 0, j)),
        ],
        out_specs=[
            pl.BlockSpec(memory_space=pltpu.SMEM),
            pl.BlockSpec(memory_space=pltpu.SMEM),
            pl.BlockSpec(memory_space=pltpu.SMEM),
        ],
        out_shape=[s11, s11, s11],
        scratch_shapes=[
            pltpu.VMEM((_B, _NP), jnp.float32),
            pltpu.SMEM((1,), jnp.float32),
            pltpu.SMEM((1,), jnp.float32),
            pltpu.SMEM((1,), jnp.float32),
        ],
    )(lt, target_classes, tbbt, pbbt)

    return total.reshape(()), loc.reshape(()), mined.reshape(())
